# Initial kernel scaffold; baseline (speedup 1.0000x reference)
import jax
import jax.numpy as jnp
from jax.experimental import pallas as pl


def kernel(feats, num_patches, patch_ids):
    raise NotImplementedError("write your pallas kernel here")


# speedup vs baseline: 4.9388x; 4.9388x over previous
import functools

import jax
import jax.numpy as jnp
from jax import lax
from jax.experimental import pallas as pl
from jax.experimental.pallas import tpu as pltpu
from jax.experimental.pallas import tpu_sc as plsc

_LANES = 16
_GROUPS = 4


def _rsqrt_newton(x):
    i = lax.bitcast_convert_type(x, jnp.int32)
    i = jnp.int32(0x5F3759DF) - lax.shift_right_logical(i, 1)
    y = lax.bitcast_convert_type(i, jnp.float32)
    for _ in range(3):
        y = y * (1.5 - 0.5 * x * y * y)
    return y


def _make_gather_norm(L, B, C, HW, NP):
    ROWS = L * B * NP
    NW = 32
    RPW = ROWS // NW
    GR = RPW // _GROUPS
    CHUNKS = C // _LANES
    mesh = plsc.VectorSubcoreMesh(core_axis_name="c", subcore_axis_name="s")

    @functools.partial(
        pl.kernel,
        out_type=jax.ShapeDtypeStruct((L, B * NP, C), jnp.float32),
        mesh=mesh,
        compiler_params=pltpu.CompilerParams(needs_layout_passes=False),
        scratch_types=(
            [pltpu.VMEM((RPW,), jnp.int32)]
            + [pltpu.VMEM((GR * C,), jnp.int32) for _ in range(_GROUPS)]
            + [pltpu.VMEM((GR * C,), jnp.float32) for _ in range(_GROUPS)]
            + [pltpu.VMEM((RPW, C), jnp.float32)]
            + [pltpu.SemaphoreType.DMA for _ in range(_GROUPS)]
        ),
    )
    def gather_norm(feats_hbm, pids_hbm, out_hbm, pid_v, *scratch):
        idx_g = scratch[:_GROUPS]
        rows_g = scratch[_GROUPS:2 * _GROUPS]
        out_blk = scratch[2 * _GROUPS]
        sems = scratch[2 * _GROUPS + 1:]

        wid = lax.axis_index("s") * 2 + lax.axis_index("c")
        r0 = wid * RPW
        g = r0 // NP
        n0 = r0 % NP
        lvl = g // B
        rl = r0 % (B * NP)
        base = g * (C * HW)

        pltpu.sync_copy(pids_hbm.at[pl.ds(lvl * NP + n0, RPW)], pid_v)

        copies = []
        for grp in range(_GROUPS):
            idx_v = idx_g[grp]

            def build(j, carry, _grp=grp, _idx=idx_v):
                p = plsc.load_gather(
                    pid_v, [jnp.full((_LANES,), _grp * GR + j, jnp.int32)])
                iota = lax.broadcasted_iota(jnp.int32, (_LANES,), 0)
                cbase = iota * HW + base + p
                off = j * C
                for k in range(CHUNKS):
                    _idx[pl.ds(off + k * _LANES, _LANES)] = (
                        cbase + (k * _LANES * HW))
                return carry

            lax.fori_loop(0, GR, build, 0, unroll=False)
            copies.append(
                pltpu.async_copy(feats_hbm.at[idx_v], rows_g[grp], sems[grp]))

        for grp in range(_GROUPS):
            copies[grp].wait()
            rows_v = rows_g[grp]

            def norm(j, carry, _grp=grp, _rows=rows_v):
                off = j * C
                acc = jnp.zeros((_LANES,), jnp.float32)
                for k in range(CHUNKS):
                    v = _rows[pl.ds(off + k * _LANES, _LANES)]
                    acc = acc + v * v
                s = jnp.full((_LANES,), jnp.sum(acc))
                y = _rsqrt_newton(s)
                d = s * y + 1e-7
                y2 = _rsqrt_newton(d)
                r = y2 * y2
                for k in range(CHUNKS):
                    out_blk[_grp * GR + j, pl.ds(k * _LANES, _LANES)] = (
                        _rows[pl.ds(off + k * _LANES, _LANES)] * r)
                return carry

            lax.fori_loop(0, GR, norm, 0, unroll=False)

        pltpu.sync_copy(out_blk, out_hbm.at[lvl, pl.ds(rl, RPW), :])

    return gather_norm


def kernel(feats, num_patches, patch_ids):
    L, B, C, H, W = feats.shape
    NP = patch_ids.shape[1]
    HW = H * W
    feats_flat = feats.reshape(-1)
    pids_flat = patch_ids.astype(jnp.int32).reshape(-1)
    out = _make_gather_norm(L, B, C, HW, NP)(feats_flat, pids_flat)
    return out, patch_ids + jnp.zeros((), patch_ids.dtype)


# speedup vs baseline: 5.0201x; 1.0165x over previous
import functools

import jax
import jax.numpy as jnp
from jax import lax
from jax.experimental import pallas as pl
from jax.experimental.pallas import tpu as pltpu
from jax.experimental.pallas import tpu_sc as plsc

_LANES = 16
_GROUPS = 2


def _rsqrt_newton(x):
    i = lax.bitcast_convert_type(x, jnp.int32)
    i = jnp.int32(0x5F3759DF) - lax.shift_right_logical(i, 1)
    y = lax.bitcast_convert_type(i, jnp.float32)
    for _ in range(3):
        y = y * (1.5 - 0.5 * x * y * y)
    return y


def _make_gather_norm(L, B, C, HW, NP):
    ROWS = L * B * NP
    NW = 32
    RPW = ROWS // NW
    GR = RPW // _GROUPS
    CHUNKS = C // _LANES
    mesh = plsc.VectorSubcoreMesh(core_axis_name="c", subcore_axis_name="s")

    @functools.partial(
        pl.kernel,
        out_type=jax.ShapeDtypeStruct((L, B * NP, C), jnp.float32),
        mesh=mesh,
        compiler_params=pltpu.CompilerParams(needs_layout_passes=False),
        scratch_types=(
            [pltpu.VMEM((RPW,), jnp.int32)]
            + [pltpu.VMEM((GR * C,), jnp.int32) for _ in range(_GROUPS)]
            + [pltpu.VMEM((GR * C,), jnp.float32) for _ in range(_GROUPS)]
            + [pltpu.VMEM((RPW, C), jnp.float32)]
            + [pltpu.SemaphoreType.DMA for _ in range(_GROUPS)]
        ),
    )
    def gather_norm(feats_hbm, pids_hbm, out_hbm, pid_v, *scratch):
        idx_g = scratch[:_GROUPS]
        rows_g = scratch[_GROUPS:2 * _GROUPS]
        out_blk = scratch[2 * _GROUPS]
        sems = scratch[2 * _GROUPS + 1:]

        wid = lax.axis_index("s") * 2 + lax.axis_index("c")
        r0 = wid * RPW
        g = r0 // NP
        n0 = r0 % NP
        lvl = g // B
        rl = r0 % (B * NP)
        base = g * (C * HW)

        pltpu.sync_copy(pids_hbm.at[pl.ds(lvl * NP + n0, RPW)], pid_v)

        copies = []
        for grp in range(_GROUPS):
            idx_v = idx_g[grp]

            def build(j, carry, _grp=grp, _idx=idx_v):
                p = plsc.load_gather(
                    pid_v, [jnp.full((_LANES,), _grp * GR + j, jnp.int32)])
                iota = lax.broadcasted_iota(jnp.int32, (_LANES,), 0)
                cbase = iota * HW + base + p
                off = j * C
                for k in range(CHUNKS):
                    _idx[pl.ds(off + k * _LANES, _LANES)] = (
                        cbase + (k * _LANES * HW))
                return carry

            lax.fori_loop(0, GR, build, 0, unroll=False)
            copies.append(
                pltpu.async_copy(feats_hbm.at[idx_v], rows_g[grp], sems[grp]))

        for grp in range(_GROUPS):
            copies[grp].wait()
            rows_v = rows_g[grp]

            def norm(j, carry, _grp=grp, _rows=rows_v):
                off = j * C
                acc = jnp.zeros((_LANES,), jnp.float32)
                for k in range(CHUNKS):
                    v = _rows[pl.ds(off + k * _LANES, _LANES)]
                    acc = acc + v * v
                s = jnp.full((_LANES,), jnp.sum(acc))
                y = _rsqrt_newton(s)
                d = s * y + 1e-7
                y2 = _rsqrt_newton(d)
                r = y2 * y2
                for k in range(CHUNKS):
                    out_blk[_grp * GR + j, pl.ds(k * _LANES, _LANES)] = (
                        _rows[pl.ds(off + k * _LANES, _LANES)] * r)
                return carry

            lax.fori_loop(0, GR, norm, 0, unroll=False)

        pltpu.sync_copy(out_blk, out_hbm.at[lvl, pl.ds(rl, RPW), :])

    return gather_norm


def kernel(feats, num_patches, patch_ids):
    L, B, C, H, W = feats.shape
    NP = patch_ids.shape[1]
    HW = H * W
    feats_flat = feats.reshape(-1)
    pids_flat = patch_ids.astype(jnp.int32).reshape(-1)
    out = _make_gather_norm(L, B, C, HW, NP)(feats_flat, pids_flat)
    return out, patch_ids + jnp.zeros((), patch_ids.dtype)


# speedup vs baseline: 5.0205x; 1.0001x over previous
import functools

import jax
import jax.numpy as jnp
from jax import lax
from jax.experimental import pallas as pl
from jax.experimental.pallas import tpu as pltpu
from jax.experimental.pallas import tpu_sc as plsc

_LANES = 16


def _rsqrt_newton(x):
    i = lax.bitcast_convert_type(x, jnp.int32)
    i = jnp.int32(0x5F3759DF) - lax.shift_right_logical(i, 1)
    y = lax.bitcast_convert_type(i, jnp.float32)
    for _ in range(3):
        y = y * (1.5 - 0.5 * x * y * y)
    return y


def _make_gather_norm(L, B, C, HW, NP):
    ROWS = L * B * NP
    NW = 32
    RPW = ROWS // NW
    CHUNKS = C // _LANES
    mesh = plsc.VectorSubcoreMesh(core_axis_name="c", subcore_axis_name="s")

    @functools.partial(
        pl.kernel,
        out_type=jax.ShapeDtypeStruct((L, B * NP, C), jnp.float32),
        mesh=mesh,
        compiler_params=pltpu.CompilerParams(needs_layout_passes=False),
        scratch_types=[
            pltpu.VMEM((RPW,), jnp.int32),
            pltpu.VMEM((RPW * C,), jnp.int32),
            pltpu.VMEM((RPW * C,), jnp.float32),
            pltpu.VMEM((RPW, C), jnp.float32),
            pltpu.SemaphoreType.DMA,
        ],
    )
    def gather_norm(feats_hbm, pids_hbm, out_hbm, pid_v, idx_v, rows_v,
                    out_blk, sem):
        wid = lax.axis_index("s") * 2 + lax.axis_index("c")
        r0 = wid * RPW
        g = r0 // NP
        n0 = r0 % NP
        lvl = g // B
        rl = r0 % (B * NP)
        base = g * (C * HW)

        pltpu.sync_copy(pids_hbm.at[lvl, pl.ds(n0, RPW)], pid_v)

        def build(j, carry):
            p = plsc.load_gather(pid_v, [jnp.full((_LANES,), j, jnp.int32)])
            iota = lax.broadcasted_iota(jnp.int32, (_LANES,), 0)
            cbase = iota * HW + base + p
            off = j * C
            for k in range(CHUNKS):
                idx_v[pl.ds(off + k * _LANES, _LANES)] = (
                    cbase + (k * _LANES * HW))
            return carry

        lax.fori_loop(0, RPW, build, 0, unroll=False)

        pltpu.async_copy(feats_hbm.at[idx_v], rows_v, sem).wait()

        def norm(j, carry):
            off = j * C
            acc = jnp.zeros((_LANES,), jnp.float32)
            for k in range(CHUNKS):
                v = rows_v[pl.ds(off + k * _LANES, _LANES)]
                acc = acc + v * v
            s = jnp.full((_LANES,), jnp.sum(acc))
            y = _rsqrt_newton(s)
            d = s * y + 1e-7
            y2 = _rsqrt_newton(d)
            r = y2 * y2
            for k in range(CHUNKS):
                out_blk[j, pl.ds(k * _LANES, _LANES)] = (
                    rows_v[pl.ds(off + k * _LANES, _LANES)] * r)
            return carry

        lax.fori_loop(0, RPW, norm, 0, unroll=False)

        pltpu.sync_copy(out_blk, out_hbm.at[lvl, pl.ds(rl, RPW), :])

    return gather_norm


def kernel(feats, num_patches, patch_ids):
    L, B, C, H, W = feats.shape
    NP = patch_ids.shape[1]
    out = _make_gather_norm(L, B, C, H * W, NP)(
        feats.reshape(-1), patch_ids.astype(jnp.int32))
    return out, patch_ids + jnp.zeros((), patch_ids.dtype)


# speedup vs baseline: 5.1183x; 1.0195x over previous
import functools

import jax
import jax.numpy as jnp
from jax import lax
from jax.experimental import pallas as pl
from jax.experimental.pallas import tpu as pltpu
from jax.experimental.pallas import tpu_sc as plsc

_LANES = 16
_GROUPS = 4


def _rsqrt_newton(x):
    i = lax.bitcast_convert_type(x, jnp.int32)
    i = jnp.int32(0x5F3759DF) - lax.shift_right_logical(i, 1)
    y = lax.bitcast_convert_type(i, jnp.float32)
    for _ in range(3):
        y = y * (1.5 - 0.5 * x * y * y)
    return y


def _make_gather_norm(L, B, C, HW, NP):
    ROWS = L * B * NP
    NW = 32
    RPW = ROWS // NW
    GR = RPW // _GROUPS
    CHUNKS = C // _LANES
    mesh = plsc.VectorSubcoreMesh(core_axis_name="c", subcore_axis_name="s")

    @functools.partial(
        pl.kernel,
        out_type=jax.ShapeDtypeStruct((ROWS * C,), jnp.float32),
        mesh=mesh,
        compiler_params=pltpu.CompilerParams(needs_layout_passes=False),
        scratch_types=(
            [pltpu.VMEM((RPW,), jnp.int32)]
            + [pltpu.VMEM((GR * C,), jnp.int32) for _ in range(_GROUPS)]
            + [pltpu.VMEM((RPW * C,), jnp.float32)]
            + [pltpu.SemaphoreType.DMA for _ in range(_GROUPS)]
        ),
    )
    def gather_norm(feats_hbm, pids_hbm, out_hbm, pid_v, *scratch):
        idx_g = scratch[:_GROUPS]
        rows_v = scratch[_GROUPS]
        sems = scratch[_GROUPS + 1:]

        wid = lax.axis_index("s") * 2 + lax.axis_index("c")
        r0 = wid * RPW
        g = r0 // NP
        n0 = r0 % NP
        lvl = g // B
        base = g * (C * HW)

        pltpu.sync_copy(pids_hbm.at[pl.ds(lvl * NP + n0, RPW)], pid_v)

        def make_build(grp):
            def build(j, carry):
                p = plsc.load_gather(
                    pid_v, [jnp.full((_LANES,), grp * GR + j, jnp.int32)])
                iota = lax.broadcasted_iota(jnp.int32, (_LANES,), 0)
                cbase = iota * HW + base + p
                off = j * C
                for k in range(CHUNKS):
                    idx_g[grp][pl.ds(off + k * _LANES, _LANES)] = (
                        cbase + (k * _LANES * HW))
                return carry
            return build

        def make_norm(grp):
            def norm(j, carry):
                off = (grp * GR + j) * C
                acc = jnp.zeros((_LANES,), jnp.float32)
                for k in range(CHUNKS):
                    v = rows_v[pl.ds(off + k * _LANES, _LANES)]
                    acc = acc + v * v
                s = jnp.full((_LANES,), jnp.sum(acc))
                y = _rsqrt_newton(s)
                d = s * y + 1e-7
                y2 = _rsqrt_newton(d)
                r = y2 * y2
                for k in range(CHUNKS):
                    rows_v[pl.ds(off + k * _LANES, _LANES)] = (
                        rows_v[pl.ds(off + k * _LANES, _LANES)] * r)
                return carry
            return norm

        copies = [None] * _GROUPS
        for grp in range(_GROUPS):
            lax.fori_loop(0, GR, make_build(grp), 0, unroll=False)
            if grp > 0:
                copies[grp - 1].wait()
            copies[grp] = pltpu.async_copy(
                feats_hbm.at[idx_g[grp]],
                rows_v.at[pl.ds(grp * GR * C, GR * C)], sems[grp])
            if grp > 0:
                lax.fori_loop(0, GR, make_norm(grp - 1), 0, unroll=False)
        copies[_GROUPS - 1].wait()
        lax.fori_loop(0, GR, make_norm(_GROUPS - 1), 0, unroll=False)

        pltpu.sync_copy(rows_v, out_hbm.at[pl.ds(r0 * C, RPW * C)])

    return gather_norm


def kernel(feats, num_patches, patch_ids):
    L, B, C, H, W = feats.shape
    NP = patch_ids.shape[1]
    HW = H * W
    feats_flat = feats.reshape(-1)
    pids_flat = patch_ids.astype(jnp.int32).reshape(-1)
    out = _make_gather_norm(L, B, C, HW, NP)(feats_flat, pids_flat)
    return out.reshape(L, B * NP, C), patch_ids + jnp.zeros((), patch_ids.dtype)


# speedup vs baseline: 5.3736x; 1.0499x over previous
import functools

import jax
import jax.numpy as jnp
from jax import lax
from jax.experimental import pallas as pl
from jax.experimental.pallas import tpu as pltpu
from jax.experimental.pallas import tpu_sc as plsc

_LANES = 16


def _rsqrt_newton(x):
    i = lax.bitcast_convert_type(x, jnp.int32)
    i = jnp.int32(0x5F3759DF) - lax.shift_right_logical(i, 1)
    y = lax.bitcast_convert_type(i, jnp.float32)
    for _ in range(3):
        y = y * (1.5 - 0.5 * x * y * y)
    return y


def _make_gather_norm(L, B, C, HW, NP):
    ROWS = L * B * NP
    NW = 32
    RPW = ROWS // NW
    CHUNKS = C // _LANES
    mesh = plsc.VectorSubcoreMesh(core_axis_name="c", subcore_axis_name="s")

    @functools.partial(
        pl.kernel,
        out_type=jax.ShapeDtypeStruct((ROWS * C,), jnp.float32),
        mesh=mesh,
        compiler_params=pltpu.CompilerParams(needs_layout_passes=False),
        scratch_types=[
            pltpu.VMEM((RPW,), jnp.int32),
            pltpu.VMEM((RPW * C,), jnp.int32),
            pltpu.VMEM((RPW * C,), jnp.float32),
            pltpu.SemaphoreType.DMA,
        ],
    )
    def gather_norm(feats_hbm, pids_hbm, out_hbm, pid_v, idx_v, rows_v, sem):
        wid = lax.axis_index("s") * 2 + lax.axis_index("c")
        r0 = wid * RPW
        g = r0 // NP
        n0 = r0 % NP
        lvl = g // B
        base = g * (C * HW)

        pltpu.sync_copy(pids_hbm.at[pl.ds(lvl * NP + n0, RPW)], pid_v)

        def build(j2, carry):
            iota = lax.broadcasted_iota(jnp.int32, (_LANES,), 0)
            for half in range(2):
                j = j2 * 2 + half
                p = plsc.load_gather(
                    pid_v, [jnp.full((_LANES,), j, jnp.int32)])
                cbase = iota * HW + base + p
                off = j * C
                for k in range(CHUNKS):
                    idx_v[pl.ds(off + k * _LANES, _LANES)] = (
                        cbase + (k * _LANES * HW))
            return carry

        lax.fori_loop(0, RPW // 2, build, 0, unroll=False)

        pltpu.async_copy(feats_hbm.at[idx_v], rows_v, sem).wait()

        def norm(j2, carry):
            offs = [(j2 * 2 + half) * C for half in range(2)]
            accs = [jnp.zeros((_LANES,), jnp.float32) for _ in range(2)]
            for k in range(CHUNKS):
                for half in range(2):
                    v = rows_v[pl.ds(offs[half] + k * _LANES, _LANES)]
                    accs[half] = accs[half] + v * v
            rs = []
            for half in range(2):
                s = jnp.full((_LANES,), jnp.sum(accs[half]))
                y = _rsqrt_newton(s)
                d = s * y + 1e-7
                y2 = _rsqrt_newton(d)
                rs.append(y2 * y2)
            for k in range(CHUNKS):
                for half in range(2):
                    rows_v[pl.ds(offs[half] + k * _LANES, _LANES)] = (
                        rows_v[pl.ds(offs[half] + k * _LANES, _LANES)]
                        * rs[half])
            return carry

        lax.fori_loop(0, RPW // 2, norm, 0, unroll=False)

        pltpu.sync_copy(rows_v, out_hbm.at[pl.ds(r0 * C, RPW * C)])

    return gather_norm


def kernel(feats, num_patches, patch_ids):
    L, B, C, H, W = feats.shape
    NP = patch_ids.shape[1]
    HW = H * W
    feats_flat = feats.reshape(-1)
    pids_flat = patch_ids.astype(jnp.int32).reshape(-1)
    out = _make_gather_norm(L, B, C, HW, NP)(feats_flat, pids_flat)
    return out.reshape(L, B * NP, C), patch_ids
